# Initial kernel scaffold; baseline (speedup 1.0000x reference)
import jax
import jax.numpy as jnp
from jax.experimental import pallas as pl


def kernel(x, edge_index, x1, edge_index_l, edge_attrs, edge_index_u, edge_attrs2, node_layers, fcnn_node, edge_layers, edge_biases, fcnn_edge):
    raise NotImplementedError("write your pallas kernel here")


# speedup vs baseline: 1.0563x; 1.0563x over previous
import functools

import jax
import jax.numpy as jnp
from jax import lax
from jax.experimental import pallas as pl
from jax.experimental.pallas import tpu as pltpu
from jax.experimental.pallas import tpu_sc as plsc

_CHUNK = 128
_NW = 32
_DEBUG_JNP_SPMM = False
_DEBUG_DENSE_REF = False


def _dedup_sorted(edge_index, n, attr=None):
    r = edge_index[0].astype(jnp.int32)
    c = edge_index[1].astype(jnp.int32)
    key = r * n + c
    order = jnp.argsort(key, stable=True)
    skey = key[order]
    first = jnp.concatenate(
        [jnp.ones((1,), jnp.bool_), skey[:-1] != skey[1:]])
    srow = skey // n
    scol = skey % n
    if attr is None:
        srow = jnp.where(first, srow, n)
        return srow, scol, None
    a = jnp.zeros((n, n), jnp.float32).at[r, c].set(attr)
    w = jnp.where(first, a[srow, scol], jnp.float32(0.0))
    return srow, scol, w


def _mm_in_body(x_ref, x1_ref, wn_ref, bn_ref, wl_ref, wu_ref,
                u_ref, vl_ref, vu_ref):
    u_ref[...] = (
        jnp.dot(x_ref[...], wn_ref[...], preferred_element_type=jnp.float32)
        + bn_ref[...])
    blk = x1_ref.shape[0]
    zpad = jnp.zeros((blk, 128 - wl_ref.shape[1]), jnp.float32)
    vl_ref[...] = jnp.concatenate(
        [jnp.dot(x1_ref[...], wl_ref[...],
                 preferred_element_type=jnp.float32), zpad], axis=1)
    vu_ref[...] = jnp.concatenate(
        [jnp.dot(x1_ref[...], wu_ref[...],
                 preferred_element_type=jnp.float32), zpad], axis=1)


def _mm_in(x, x1, wn, bn, wl, wu):
    n, dn = x.shape
    ne, de = x1.shape
    blk = 512
    grid = (n // blk,)
    full = lambda i: (0, 0)
    return pl.pallas_call(
        _mm_in_body,
        grid=grid,
        in_specs=[
            pl.BlockSpec((blk, dn), lambda i: (i, 0)),
            pl.BlockSpec((blk, de), lambda i: (i, 0)),
            pl.BlockSpec(wn.shape, full),
            pl.BlockSpec(bn.shape, full),
            pl.BlockSpec(wl.shape, full),
            pl.BlockSpec(wu.shape, full),
        ],
        out_specs=[
            pl.BlockSpec((blk, wn.shape[1]), lambda i: (i, 0)),
            pl.BlockSpec((blk, 128), lambda i: (i, 0)),
            pl.BlockSpec((blk, 128), lambda i: (i, 0)),
        ],
        out_shape=[
            jax.ShapeDtypeStruct((n, wn.shape[1]), jnp.float32),
            jax.ShapeDtypeStruct((ne, 128), jnp.float32),
            jax.ShapeDtypeStruct((ne, 128), jnp.float32),
        ],
    )(x, x1, wn, bn, wl, wu)


def _spmm_body(de_real, u_hbm, vl_hbm, vu_hbm,
               rn_hbm, cn_hbm,
               rl_hbm, cl_hbm, wl_hbm,
               ru_hbm, cu_hbm, wu_hbm,
               yn_hbm, ye_hbm,
               acc_n, acc_e, gbuf_n, gbuf_e, cidx, ridx, wbuf):
    cid = lax.axis_index("c")
    sid = lax.axis_index("s")
    n = acc_e.shape[0]
    dn = gbuf_n.shape[1]
    de = gbuf_e.shape[1]
    rows_per_tile = n // 16
    e_per_tile = rn_hbm.shape[0] // _NW

    def zrow(i, c):
        z16 = jnp.zeros((16,), jnp.float32)
        for j in range(dn // 16):
            gbuf_n[i, pl.ds(j * 16, 16)] = z16
        for j in range(de // 16):
            gbuf_e[i, pl.ds(j * 16, 16)] = z16
        return c
    lax.fori_loop(0, _CHUNK, zrow, 0)
    for t in range(rows_per_tile // _CHUNK):
        base = sid * rows_per_tile + t * _CHUNK
        pltpu.sync_copy(gbuf_n, acc_n.at[pl.ds(base, _CHUNK)])
        pltpu.sync_copy(gbuf_e, acc_e.at[pl.ds(base, _CHUNK)])
    plsc.subcore_barrier()

    ebase = (cid * 16 + sid) * e_per_tile

    def node_chunk(k, c):
        off = pl.multiple_of(ebase + k * _CHUNK, _CHUNK)
        pltpu.sync_copy(cn_hbm.at[pl.ds(off, _CHUNK)], cidx)
        pltpu.sync_copy(rn_hbm.at[pl.ds(off, _CHUNK)], ridx)
        pltpu.sync_copy(u_hbm.at[cidx], gbuf_n)
        pltpu.sync_copy(gbuf_n, acc_n.at[ridx], add=True)
        return c
    lax.fori_loop(0, e_per_tile // _CHUNK, node_chunk, 0)

    def make_edge_chunk(r_hbm, c_hbm, w_hbm, v_hbm):
        def edge_chunk(k, c):
            off = pl.multiple_of(ebase + k * _CHUNK, _CHUNK)
            pltpu.sync_copy(c_hbm.at[pl.ds(off, _CHUNK)], cidx)
            pltpu.sync_copy(r_hbm.at[pl.ds(off, _CHUNK)], ridx)
            pltpu.sync_copy(w_hbm.at[pl.ds(off, _CHUNK)], wbuf)
            pltpu.sync_copy(v_hbm.at[cidx], gbuf_e)
            def scale(g, cc):
                w16 = wbuf[pl.ds(g * 16, 16)]
                for j in range(16):
                    w = w16[j]
                    row = g * 16 + j
                    for q in range(de_real // 16):
                        gbuf_e[row, pl.ds(q * 16, 16)] = (
                            gbuf_e[row, pl.ds(q * 16, 16)] * w)
                return cc
            lax.fori_loop(0, _CHUNK // 16, scale, 0)
            pltpu.sync_copy(gbuf_e, acc_e.at[ridx], add=True)
            return c
        return edge_chunk
    lax.fori_loop(0, e_per_tile // _CHUNK,
                  make_edge_chunk(rl_hbm, cl_hbm, wl_hbm, vl_hbm), 0)
    lax.fori_loop(0, e_per_tile // _CHUNK,
                  make_edge_chunk(ru_hbm, cu_hbm, wu_hbm, vu_hbm), 0)
    plsc.subcore_barrier()

    out_base = cid * n + sid * rows_per_tile
    pltpu.sync_copy(acc_n.at[pl.ds(sid * rows_per_tile, rows_per_tile)],
                    yn_hbm.at[pl.ds(out_base, rows_per_tile)])
    pltpu.sync_copy(acc_e.at[pl.ds(sid * rows_per_tile, rows_per_tile)],
                    ye_hbm.at[pl.ds(out_base, rows_per_tile)])


def _spmm(u, vl, vu, rn, cn, rl, cl, wl, ru, cu, wu):
    n, dn = u.shape
    de = vl.shape[1]
    mesh = plsc.VectorSubcoreMesh(core_axis_name="c", subcore_axis_name="s")
    f = pl.kernel(
        functools.partial(_spmm_body, 64),
        out_type=[
            jax.ShapeDtypeStruct((2 * n, dn), jnp.float32),
            jax.ShapeDtypeStruct((2 * n, de), jnp.float32),
        ],
        mesh=mesh,
        scratch_types=[
            pltpu.VMEM_SHARED((n + 1, dn), jnp.float32),
            pltpu.VMEM_SHARED((n, de), jnp.float32),
            pltpu.VMEM((_CHUNK, dn), jnp.float32),
            pltpu.VMEM((_CHUNK, de), jnp.float32),
            pltpu.VMEM((_CHUNK,), jnp.int32),
            pltpu.VMEM((_CHUNK,), jnp.int32),
            pltpu.VMEM((_CHUNK,), jnp.float32),
        ],
    )
    return f(u, vl, vu, rn, cn, rl, cl, wl, ru, cu, wu)


def _mm_out_body(yn_ref, ye_ref, wf_ref, bf_ref, be_ref, wfe_ref, bfe_ref,
                 node_ref, edge_ref):
    h = jax.nn.relu(yn_ref[0] + yn_ref[1])
    node_ref[...] = (
        jnp.dot(h, wf_ref[...], preferred_element_type=jnp.float32)
        + bf_ref[...])
    de = be_ref.shape[1]
    he = jax.nn.relu(ye_ref[0, :, :de] + ye_ref[1, :, :de] + be_ref[...])
    edge_ref[...] = (
        jnp.dot(he, wfe_ref[...], preferred_element_type=jnp.float32)
        + bfe_ref[...])


def _mm_out(yn, ye, wf, bf, be, wfe, bfe):
    n = yn.shape[1]
    dn = yn.shape[2]
    de = ye.shape[2]
    blk = 512
    grid = (n // blk,)
    full = lambda i: (0, 0)
    return pl.pallas_call(
        _mm_out_body,
        grid=grid,
        in_specs=[
            pl.BlockSpec((2, blk, dn), lambda i: (0, i, 0)),
            pl.BlockSpec((2, blk, de), lambda i: (0, i, 0)),
            pl.BlockSpec(wf.shape, full),
            pl.BlockSpec(bf.shape, full),
            pl.BlockSpec(be.shape, full),
            pl.BlockSpec(wfe.shape, full),
            pl.BlockSpec(bfe.shape, full),
        ],
        out_specs=[
            pl.BlockSpec((blk, wf.shape[1]), lambda i: (i, 0)),
            pl.BlockSpec((blk, wfe.shape[1]), lambda i: (i, 0)),
        ],
        out_shape=[
            jax.ShapeDtypeStruct((n, wf.shape[1]), jnp.float32),
            jax.ShapeDtypeStruct((n, wfe.shape[1]), jnp.float32),
        ],
    )(yn, ye, wf, bf, be, wfe, bfe)


def kernel(x, edge_index, x1, edge_index_l, edge_attrs, edge_index_u,
           edge_attrs2, node_layers, fcnn_node, edge_layers, edge_biases,
           fcnn_edge):
    n = x.shape[0]
    ne = x1.shape[0]

    hops_n = node_layers[0]
    wn = sum(lin["W"] for lin in hops_n)
    bn = sum(lin["b"] for lin in hops_n).reshape(1, -1)
    hops_e = edge_layers[0]
    wl = sum(pair[0]["W"] for pair in hops_e)
    wu = sum(pair[1]["W"] for pair in hops_e)
    be = edge_biases[0].reshape(1, -1)

    rn, cn, _ = _dedup_sorted(edge_index, n, None)
    rl, cl, wlv = _dedup_sorted(edge_index_l, ne, edge_attrs)
    ru, cu, wuv = _dedup_sorted(edge_index_u, ne, edge_attrs2)

    if _DEBUG_DENSE_REF:
        A_x = jnp.zeros((n, n)).at[edge_index[0], edge_index[1]].set(1.0)
        L_l = jnp.zeros((ne, ne)).at[edge_index_l[0],
                                     edge_index_l[1]].set(edge_attrs)
        L_u = jnp.zeros((ne, ne)).at[edge_index_u[0],
                                     edge_index_u[1]].set(edge_attrs2)
        h = jax.nn.relu(A_x @ (x @ wn + bn))
        node_out = h @ fcnn_node["W"] + fcnn_node["b"]
        he = jax.nn.relu(L_l @ (x1 @ wl) + L_u @ (x1 @ wu) + be)
        edge_out = he @ fcnn_edge["W"] + fcnn_edge["b"]
        return node_out, edge_out

    u, vl, vu = _mm_in(x, x1, wn, bn, wl, wu)
    if _DEBUG_JNP_SPMM:
        yn0 = jnp.zeros((n + 1, u.shape[1])).at[rn].add(u[cn])[:n]
        ye0 = (jnp.zeros((ne, vl.shape[1]))
               .at[rl].add(vl[cl] * wlv[:, None])
               .at[ru].add(vu[cu] * wuv[:, None]))
        yn = jnp.stack([yn0, jnp.zeros_like(yn0)])
        ye = jnp.stack([ye0, jnp.zeros_like(ye0)])
    else:
        yn, ye = _spmm(u, vl, vu, rn, cn, rl, cl, wlv, ru, cu, wuv)
        yn = yn.reshape(2, n, -1)
        ye = ye.reshape(2, ne, -1)

    node_out, edge_out = _mm_out(
        yn, ye, fcnn_node["W"], fcnn_node["b"].reshape(1, -1), be,
        fcnn_edge["W"], fcnn_edge["b"].reshape(1, -1))
    return node_out, edge_out
